# baseline (speedup 1.0000x reference)
import functools

import jax
import jax.numpy as jnp
from jax import lax
from jax.experimental import pallas as pl
from jax.experimental.pallas import tpu as pltpu
from jax.experimental.pallas import tpu_sc as plsc

SUPPORT_RANGE = 300
NUM_BINS = 2 * SUPPORT_RANGE + 1

N_ROWS = 16384
NUM_CORES = 2
NUM_SUBCORES = 16
NUM_WORKERS = NUM_CORES * NUM_SUBCORES
ROWS_PER_WORKER = N_ROWS // NUM_WORKERS
CHUNK = 64
N_CHUNKS = ROWS_PER_WORKER // CHUNK
GROUPS = CHUNK // 16
CHUNK_ELEMS = CHUNK * NUM_BINS


def _sc_body(logits_hbm, out_hbm, in_v, out_v):
    wid = lax.axis_index("s") * NUM_CORES + lax.axis_index("c")
    base = wid * ROWS_PER_WORKER * NUM_BINS

    lane = lax.iota(jnp.int32, 16)
    lane_off = lane * NUM_BINS
    zeros = jnp.zeros((16,), jnp.float32)
    ones16 = jnp.full((16,), 1.0, jnp.float32)
    neg_sup = jnp.full((16,), -float(SUPPORT_RANGE), jnp.float32)

    def chunk_body(ci, carry):
        elem0 = base + ci * CHUNK_ELEMS
        pltpu.sync_copy(logits_hbm.at[pl.ds(elem0, CHUNK_ELEMS)], in_v)

        def group_body(g, carry):
            rbase = g * (16 * NUM_BINS) + lane_off

            def max_body(c, m):
                idx = rbase + c
                v = plsc.load_gather(in_v, [idx])
                return jnp.maximum(m, v)

            m = lax.fori_loop(
                0, NUM_BINS, max_body, jnp.full((16,), -jnp.inf, jnp.float32)
            )

            def sum_body(c, acc):
                s0, s1, sup = acc
                v = plsc.load_gather(in_v, [rbase + c])
                e = jnp.exp(v - m)
                return (s0 + e, s1 + e * sup, sup + ones16)

            s0, s1, _ = lax.fori_loop(0, NUM_BINS, sum_body, (zeros, zeros, neg_sup))
            x = s1 / s0
            y = jnp.clip(x, -float(SUPPORT_RANGE), float(SUPPORT_RANGE))

            def tent_body(c, sup):
                w = jnp.maximum(0.0, 1.0 - jnp.abs(sup - y))
                plsc.store_scatter(out_v, [rbase + c], w)
                return sup + ones16

            lax.fori_loop(0, NUM_BINS, tent_body, neg_sup)
            return carry

        lax.fori_loop(0, GROUPS, group_body, 0)
        pltpu.sync_copy(out_v, out_hbm.at[pl.ds(elem0, CHUNK_ELEMS)])
        return carry

    lax.fori_loop(0, N_CHUNKS, chunk_body, 0)


@jax.jit
def kernel(logits):
    mesh = plsc.VectorSubcoreMesh(core_axis_name="c", subcore_axis_name="s")
    run = functools.partial(
        pl.kernel,
        mesh=mesh,
        out_type=jax.ShapeDtypeStruct((N_ROWS * NUM_BINS,), jnp.float32),
        scratch_types=[
            pltpu.VMEM((CHUNK_ELEMS,), jnp.float32),
            pltpu.VMEM((CHUNK_ELEMS,), jnp.float32),
        ],
        compiler_params=pltpu.CompilerParams(needs_layout_passes=False),
    )(_sc_body)
    return run(logits.reshape(-1)).reshape(N_ROWS, NUM_BINS)


# speedup vs baseline: 1.6760x; 1.6760x over previous
import functools

import jax
import jax.numpy as jnp
from jax import lax
from jax.experimental import pallas as pl
from jax.experimental.pallas import tpu as pltpu
from jax.experimental.pallas import tpu_sc as plsc

SUPPORT_RANGE = 300
NUM_BINS = 2 * SUPPORT_RANGE + 1

N_ROWS = 16384
NUM_CORES = 2
NUM_SUBCORES = 16
NUM_WORKERS = NUM_CORES * NUM_SUBCORES
ROWS_PER_WORKER = N_ROWS // NUM_WORKERS
CHUNK = 64
N_CHUNKS = ROWS_PER_WORKER // CHUNK
GROUPS = CHUNK // 16
CHUNK_ELEMS = CHUNK * NUM_BINS
UNROLL = 8
MAIN_COLS = (NUM_BINS - 1) // UNROLL


def _sc_body(logits_hbm, out_hbm, in_v, out_v):
    wid = lax.axis_index("s") * NUM_CORES + lax.axis_index("c")
    base = wid * ROWS_PER_WORKER * NUM_BINS

    lane = lax.iota(jnp.int32, 16)
    lane_off = lane * NUM_BINS
    zeros = jnp.zeros((16,), jnp.float32)
    fullrange = jnp.full((16,), float(SUPPORT_RANGE), jnp.float32)

    def zero_body(i, carry):
        for u in range(4):
            out_v[pl.ds((i * 4 + u) * 16, 16)] = zeros
        return carry

    lax.fori_loop(0, NUM_BINS, zero_body, 0)

    def chunk_body(ci, carry):
        elem0 = base + ci * CHUNK_ELEMS
        pltpu.sync_copy(logits_hbm.at[pl.ds(elem0, CHUNK_ELEMS)], in_v)

        lows = []
        highs = []
        for g in range(GROUPS):
            rbase = g * (16 * NUM_BINS) + lane_off

            def max_body(i, acc):
                m, idx = acc
                for _ in range(UNROLL):
                    m = jnp.maximum(m, plsc.load_gather(in_v, [idx]))
                    idx = idx + 1
                return m, idx

            m, _ = lax.fori_loop(
                0, MAIN_COLS, max_body, (jnp.full((16,), -jnp.inf, jnp.float32), rbase)
            )
            m = jnp.maximum(m, plsc.load_gather(in_v, [rbase + (NUM_BINS - 1)]))

            def sum_body(i, acc):
                s0, s1, sup, idx = acc
                for _ in range(UNROLL):
                    e = jnp.exp(plsc.load_gather(in_v, [idx]) - m)
                    s0 = s0 + e
                    s1 = s1 + e * sup
                    sup = sup + 1.0
                    idx = idx + 1
                return s0, s1, sup, idx

            s0, s1, _, _ = lax.fori_loop(
                0, MAIN_COLS, sum_body, (zeros, zeros, -fullrange, rbase)
            )
            e_last = jnp.exp(plsc.load_gather(in_v, [rbase + (NUM_BINS - 1)]) - m)
            s0 = s0 + e_last
            s1 = s1 + e_last * fullrange

            x = s1 / s0
            y = jnp.clip(x, -float(SUPPORT_RANGE), float(SUPPORT_RANGE))
            t = y + float(SUPPORT_RANGE)
            ti = jnp.clip(t.astype(jnp.int32), 0, NUM_BINS - 1)
            frac = t - ti.astype(jnp.float32)
            ilow = rbase + ti
            ihigh = rbase + jnp.minimum(ti + 1, NUM_BINS - 1)
            plsc.store_scatter(out_v, [ilow], 1.0 - frac)
            plsc.addupdate_scatter(out_v, [ihigh], frac)
            lows.append(ilow)
            highs.append(ihigh)

        pltpu.sync_copy(out_v, out_hbm.at[pl.ds(elem0, CHUNK_ELEMS)])

        for g in range(GROUPS):
            plsc.store_scatter(out_v, [lows[g]], zeros)
            plsc.store_scatter(out_v, [highs[g]], zeros)
        return carry

    lax.fori_loop(0, N_CHUNKS, chunk_body, 0)


@jax.jit
def kernel(logits):
    mesh = plsc.VectorSubcoreMesh(core_axis_name="c", subcore_axis_name="s")
    run = functools.partial(
        pl.kernel,
        mesh=mesh,
        out_type=jax.ShapeDtypeStruct((N_ROWS * NUM_BINS,), jnp.float32),
        scratch_types=[
            pltpu.VMEM((CHUNK_ELEMS,), jnp.float32),
            pltpu.VMEM((CHUNK_ELEMS,), jnp.float32),
        ],
        compiler_params=pltpu.CompilerParams(needs_layout_passes=False),
    )(_sc_body)
    return run(logits.reshape(-1)).reshape(N_ROWS, NUM_BINS)


# speedup vs baseline: 1.7969x; 1.0722x over previous
import functools

import jax
import jax.numpy as jnp
from jax import lax
from jax.experimental import pallas as pl
from jax.experimental.pallas import tpu as pltpu
from jax.experimental.pallas import tpu_sc as plsc

SUPPORT_RANGE = 300
NUM_BINS = 2 * SUPPORT_RANGE + 1

N_ROWS = 16384
NUM_CORES = 2
NUM_SUBCORES = 16
NUM_WORKERS = NUM_CORES * NUM_SUBCORES
ROWS_PER_WORKER = N_ROWS // NUM_WORKERS
CHUNK = 32
N_CHUNKS = ROWS_PER_WORKER // CHUNK
GROUPS = CHUNK // 16
CHUNK_ELEMS = CHUNK * NUM_BINS
UNROLL = 24
MAIN_COLS = (NUM_BINS - 1) // UNROLL


def _sc_body(logits_hbm, out_hbm, in0_v, in1_v, out_v, sem0, sem1):
    wid = lax.axis_index("s") * NUM_CORES + lax.axis_index("c")
    base = wid * ROWS_PER_WORKER * NUM_BINS

    lane = lax.iota(jnp.int32, 16)
    lane_off = lane * NUM_BINS
    zeros = jnp.zeros((16,), jnp.float32)
    fullrange = jnp.full((16,), float(SUPPORT_RANGE), jnp.float32)

    def zero_body(i, carry):
        for u in range(2):
            out_v[pl.ds((i * 2 + u) * 16, 16)] = zeros
        return carry

    lax.fori_loop(0, NUM_BINS, zero_body, 0)

    def in_slice(ci):
        return logits_hbm.at[pl.ds(base + ci * CHUNK_ELEMS, CHUNK_ELEMS)]

    def process_chunk(ci, in_v):
        touched = []
        for g in range(GROUPS):
            rbase = g * (16 * NUM_BINS) + lane_off

            def max_body(i, acc):
                m, idx = acc
                for _ in range(UNROLL):
                    m = jnp.maximum(m, plsc.load_gather(in_v, [idx]))
                    idx = idx + 1
                return m, idx

            m, _ = lax.fori_loop(
                0, MAIN_COLS, max_body, (jnp.full((16,), -jnp.inf, jnp.float32), rbase)
            )
            m = jnp.maximum(m, plsc.load_gather(in_v, [rbase + (NUM_BINS - 1)]))

            def sum_body(i, acc):
                s0, s1, sup, idx = acc
                for _ in range(UNROLL):
                    e = jnp.exp(plsc.load_gather(in_v, [idx]) - m)
                    s0 = s0 + e
                    s1 = s1 + e * sup
                    sup = sup + 1.0
                    idx = idx + 1
                return s0, s1, sup, idx

            s0, s1, _, _ = lax.fori_loop(
                0, MAIN_COLS, sum_body, (zeros, zeros, -fullrange, rbase)
            )
            e_last = jnp.exp(plsc.load_gather(in_v, [rbase + (NUM_BINS - 1)]) - m)
            s0 = s0 + e_last
            s1 = s1 + e_last * fullrange

            x = s1 / s0
            y = jnp.clip(x, -float(SUPPORT_RANGE), float(SUPPORT_RANGE))
            t = y + float(SUPPORT_RANGE)
            ti = jnp.clip(t.astype(jnp.int32), 0, NUM_BINS - 1)
            frac = t - ti.astype(jnp.float32)
            ilow = rbase + ti
            ihigh = rbase + jnp.minimum(ti + 1, NUM_BINS - 1)
            plsc.store_scatter(out_v, [ilow], 1.0 - frac)
            plsc.addupdate_scatter(out_v, [ihigh], frac)
            touched.append((ilow, ihigh))

        pltpu.sync_copy(out_v, out_hbm.at[pl.ds(base + ci * CHUNK_ELEMS, CHUNK_ELEMS)])

        for ilow, ihigh in touched:
            plsc.store_scatter(out_v, [ilow], zeros)
            plsc.store_scatter(out_v, [ihigh], zeros)

    pltpu.async_copy(in_slice(0), in0_v, sem0)

    def pair_body(p, carry):
        c0 = p * 2
        pltpu.make_async_copy(in_slice(c0), in0_v, sem0).wait()
        pltpu.async_copy(in_slice(c0 + 1), in1_v, sem1)
        process_chunk(c0, in0_v)
        pltpu.make_async_copy(in_slice(c0 + 1), in1_v, sem1).wait()
        nxt = jnp.minimum(c0 + 2, N_CHUNKS - 2)
        pltpu.async_copy(in_slice(nxt), in0_v, sem0)
        process_chunk(c0 + 1, in1_v)
        return carry

    lax.fori_loop(0, N_CHUNKS // 2, pair_body, 0)
    pltpu.make_async_copy(in_slice(N_CHUNKS - 2), in0_v, sem0).wait()


@jax.jit
def kernel(logits):
    mesh = plsc.VectorSubcoreMesh(core_axis_name="c", subcore_axis_name="s")
    run = functools.partial(
        pl.kernel,
        mesh=mesh,
        out_type=jax.ShapeDtypeStruct((N_ROWS * NUM_BINS,), jnp.float32),
        scratch_types=[
            pltpu.VMEM((CHUNK_ELEMS,), jnp.float32),
            pltpu.VMEM((CHUNK_ELEMS,), jnp.float32),
            pltpu.VMEM((CHUNK_ELEMS,), jnp.float32),
            pltpu.SemaphoreType.DMA,
            pltpu.SemaphoreType.DMA,
        ],
        compiler_params=pltpu.CompilerParams(needs_layout_passes=False),
    )(_sc_body)
    return run(logits.reshape(-1)).reshape(N_ROWS, NUM_BINS)


# speedup vs baseline: 2.2609x; 1.2582x over previous
import functools

import jax
import jax.numpy as jnp
from jax import lax
from jax.experimental import pallas as pl
from jax.experimental.pallas import tpu as pltpu
from jax.experimental.pallas import tpu_sc as plsc

SUPPORT_RANGE = 300
EPS = 0.001
NUM_BINS = 2 * SUPPORT_RANGE + 1

N_ROWS = 16384
BLOCK_ROWS = 2048

NUM_CORES = 2
NUM_SUBCORES = 16
NUM_WORKERS = NUM_CORES * NUM_SUBCORES
ROWS_PER_WORKER = N_ROWS // NUM_WORKERS
CHUNK = 128
N_CHUNKS = ROWS_PER_WORKER // CHUNK
GROUPS = CHUNK // 16
CHUNK_ELEMS = CHUNK * NUM_BINS


def _row_scalar_block(logits_ref, y_ref):
    logits = logits_ref[...]
    rows = logits.shape[0]

    bins = jax.lax.broadcasted_iota(jnp.int32, (rows, NUM_BINS), 1)
    support = bins.astype(jnp.float32) - float(SUPPORT_RANGE)

    m = jnp.max(logits, axis=-1, keepdims=True)
    e = jnp.exp(logits - m)
    x = jnp.sum(e * support, axis=-1, keepdims=True) / jnp.sum(
        e, axis=-1, keepdims=True
    )

    scalar = jnp.sign(x) * (
        ((jnp.sqrt(1.0 + 4.0 * EPS * (jnp.abs(x) + 1.0 + EPS)) - 1.0) / (2.0 * EPS))
        ** 2
        - 1.0
    )
    y = jnp.sign(scalar) * (jnp.sqrt(jnp.abs(scalar) + 1.0) - 1.0) + EPS * scalar
    y_ref[...] = jnp.clip(y, -float(SUPPORT_RANGE), float(SUPPORT_RANGE))


def _sc_scatter_body(y_hbm, out_hbm, y_v, out_v):
    wid = lax.axis_index("s") * NUM_CORES + lax.axis_index("c")
    row_base = wid * ROWS_PER_WORKER
    elem_base = row_base * NUM_BINS

    lane = lax.iota(jnp.int32, 16)
    lane_off = lane * NUM_BINS
    zeros = jnp.zeros((16,), jnp.float32)

    def zero_body(i, carry):
        for u in range(8):
            out_v[pl.ds((i * 8 + u) * 16, 16)] = zeros
        return carry

    lax.fori_loop(0, CHUNK_ELEMS // 128, zero_body, 0)
    rem = (CHUNK_ELEMS // 128) * 128
    for u in range((CHUNK_ELEMS - rem) // 16):
        out_v[pl.ds(rem + u * 16, 16)] = zeros

    def chunk_body(ci, carry):
        pltpu.sync_copy(y_hbm.at[pl.ds(row_base + ci * CHUNK, CHUNK)], y_v)

        touched = []
        for g in range(GROUPS):
            rbase = g * (16 * NUM_BINS) + lane_off
            y = y_v[pl.ds(g * 16, 16)]
            t = y + float(SUPPORT_RANGE)
            ti = jnp.clip(t.astype(jnp.int32), 0, NUM_BINS - 1)
            frac = t - ti.astype(jnp.float32)
            ilow = rbase + ti
            ihigh = rbase + jnp.minimum(ti + 1, NUM_BINS - 1)
            plsc.store_scatter(out_v, [ilow], 1.0 - frac)
            plsc.addupdate_scatter(out_v, [ihigh], frac)
            touched.append((ilow, ihigh))

        pltpu.sync_copy(
            out_v, out_hbm.at[pl.ds(elem_base + ci * CHUNK_ELEMS, CHUNK_ELEMS)]
        )

        for ilow, ihigh in touched:
            plsc.store_scatter(out_v, [ilow], zeros)
            plsc.store_scatter(out_v, [ihigh], zeros)
        return carry

    lax.fori_loop(0, N_CHUNKS, chunk_body, 0)


@jax.jit
def kernel(logits):
    y = pl.pallas_call(
        _row_scalar_block,
        grid=(N_ROWS // BLOCK_ROWS,),
        in_specs=[pl.BlockSpec((BLOCK_ROWS, NUM_BINS), lambda i: (i, 0))],
        out_specs=pl.BlockSpec((BLOCK_ROWS, 1), lambda i: (i, 0)),
        out_shape=jax.ShapeDtypeStruct((N_ROWS, 1), jnp.float32),
    )(logits)

    mesh = plsc.VectorSubcoreMesh(core_axis_name="c", subcore_axis_name="s")
    scatter = functools.partial(
        pl.kernel,
        mesh=mesh,
        out_type=jax.ShapeDtypeStruct((N_ROWS * NUM_BINS,), jnp.float32),
        scratch_types=[
            pltpu.VMEM((CHUNK,), jnp.float32),
            pltpu.VMEM((CHUNK_ELEMS,), jnp.float32),
        ],
        compiler_params=pltpu.CompilerParams(needs_layout_passes=False),
    )(_sc_scatter_body)
    return scatter(y.reshape(-1)).reshape(N_ROWS, NUM_BINS)


# speedup vs baseline: 2.8152x; 1.2452x over previous
import functools

import jax
import jax.numpy as jnp
from jax import lax
from jax.experimental import pallas as pl
from jax.experimental.pallas import tpu as pltpu
from jax.experimental.pallas import tpu_sc as plsc

SUPPORT_RANGE = 300
EPS = 0.001
NUM_BINS = 2 * SUPPORT_RANGE + 1

N_ROWS = 16384
BLOCK_ROWS = 2048

NUM_CORES = 2
NUM_SUBCORES = 16
NUM_WORKERS = NUM_CORES * NUM_SUBCORES
ROWS_PER_WORKER = N_ROWS // NUM_WORKERS
CHUNK = 128
N_CHUNKS = ROWS_PER_WORKER // CHUNK
GROUPS = CHUNK // 16
FULL16 = NUM_BINS // 16


def _row_scalar_block(logits_ref, y_ref):
    logits = logits_ref[...]
    rows = logits.shape[0]

    bins = jax.lax.broadcasted_iota(jnp.int32, (rows, NUM_BINS), 1)
    support = bins.astype(jnp.float32) - float(SUPPORT_RANGE)

    m = jnp.max(logits, axis=-1, keepdims=True)
    e = jnp.exp(logits - m)
    x = jnp.sum(e * support, axis=-1, keepdims=True) / jnp.sum(
        e, axis=-1, keepdims=True
    )

    scalar = jnp.sign(x) * (
        ((jnp.sqrt(1.0 + 4.0 * EPS * (jnp.abs(x) + 1.0 + EPS)) - 1.0) / (2.0 * EPS))
        ** 2
        - 1.0
    )
    y = jnp.sign(scalar) * (jnp.sqrt(jnp.abs(scalar) + 1.0) - 1.0) + EPS * scalar
    y_ref[...] = jnp.clip(y, -float(SUPPORT_RANGE), float(SUPPORT_RANGE))


def _zero_fill(out_v):
    zeros = jnp.zeros((16,), jnp.float32)

    def zero_row(r, carry):
        for u in range(FULL16):
            out_v[r, pl.ds(u * 16, 16)] = zeros
        out_v[r, pl.ds(NUM_BINS - 16, 16)] = zeros
        return carry

    lax.fori_loop(0, CHUNK, zero_row, 0)


def _sc_scatter_body(y_hbm, out_hbm, y_v, out_v):
    wid = lax.axis_index("s") * NUM_CORES + lax.axis_index("c")
    row_base = wid * ROWS_PER_WORKER

    lane = lax.iota(jnp.int32, 16)
    zeros = jnp.zeros((16,), jnp.float32)

    _zero_fill(out_v)

    def chunk_body(ci, carry):
        pltpu.sync_copy(y_hbm.at[pl.ds(row_base + ci * CHUNK, CHUNK)], y_v)

        touched = []
        for g in range(GROUPS):
            rid = g * 16 + lane
            y = y_v[pl.ds(g * 16, 16)]
            t = y + float(SUPPORT_RANGE)
            ti = jnp.clip(t.astype(jnp.int32), 0, NUM_BINS - 1)
            frac = t - ti.astype(jnp.float32)
            ihigh = jnp.minimum(ti + 1, NUM_BINS - 1)
            plsc.store_scatter(out_v, [rid, ti], 1.0 - frac)
            plsc.addupdate_scatter(out_v, [rid, ihigh], frac)
            touched.append((rid, ti, ihigh))

        pltpu.sync_copy(out_v, out_hbm.at[pl.ds(row_base + ci * CHUNK, CHUNK)])

        for rid, ti, ihigh in touched:
            plsc.store_scatter(out_v, [rid, ti], zeros)
            plsc.store_scatter(out_v, [rid, ihigh], zeros)
        return carry

    lax.fori_loop(0, N_CHUNKS, chunk_body, 0)


@jax.jit
def kernel(logits):
    y = pl.pallas_call(
        _row_scalar_block,
        grid=(N_ROWS // BLOCK_ROWS,),
        in_specs=[pl.BlockSpec((BLOCK_ROWS, NUM_BINS), lambda i: (i, 0))],
        out_specs=pl.BlockSpec((BLOCK_ROWS, 1), lambda i: (i, 0)),
        out_shape=jax.ShapeDtypeStruct((N_ROWS, 1), jnp.float32),
    )(logits)

    mesh = plsc.VectorSubcoreMesh(core_axis_name="c", subcore_axis_name="s")
    scatter = functools.partial(
        pl.kernel,
        mesh=mesh,
        out_type=jax.ShapeDtypeStruct((N_ROWS, NUM_BINS), jnp.float32),
        scratch_types=[
            pltpu.VMEM((CHUNK,), jnp.float32),
            pltpu.VMEM((CHUNK, NUM_BINS), jnp.float32),
        ],
        compiler_params=pltpu.CompilerParams(needs_layout_passes=False),
    )(_sc_scatter_body)
    return scatter(y.reshape(-1))


# speedup vs baseline: 2.9258x; 1.0393x over previous
import functools

import jax
import jax.numpy as jnp
from jax import lax
from jax.experimental import pallas as pl
from jax.experimental.pallas import tpu as pltpu
from jax.experimental.pallas import tpu_sc as plsc

SUPPORT_RANGE = 300
EPS = 0.001
NUM_BINS = 2 * SUPPORT_RANGE + 1

N_ROWS = 16384
BLOCK_ROWS = 2048

NUM_CORES = 2
NUM_SUBCORES = 16
NUM_WORKERS = NUM_CORES * NUM_SUBCORES
ROWS_PER_WORKER = N_ROWS // NUM_WORKERS
CHUNK = 128
N_CHUNKS = ROWS_PER_WORKER // CHUNK
GROUPS = CHUNK // 16
FULL16 = NUM_BINS // 16


def _row_scalar_block(logits_ref, y_ref):
    logits = logits_ref[...]
    rows = logits.shape[0]

    bins = jax.lax.broadcasted_iota(jnp.int32, (rows, NUM_BINS), 1)
    support = bins.astype(jnp.float32) - float(SUPPORT_RANGE)

    m = jnp.max(logits, axis=-1, keepdims=True)
    e = jnp.exp(logits - m)
    x = jnp.sum(e * support, axis=-1, keepdims=True) / jnp.sum(
        e, axis=-1, keepdims=True
    )

    scalar = jnp.sign(x) * (
        ((jnp.sqrt(1.0 + 4.0 * EPS * (jnp.abs(x) + 1.0 + EPS)) - 1.0) / (2.0 * EPS))
        ** 2
        - 1.0
    )
    y = jnp.sign(scalar) * (jnp.sqrt(jnp.abs(scalar) + 1.0) - 1.0) + EPS * scalar
    y = jnp.clip(y, -float(SUPPORT_RANGE), float(SUPPORT_RANGE))
    y_ref[...] = y.reshape(rows // 128, 128)


def _zero_fill(out_v):
    zeros = jnp.zeros((16,), jnp.float32)

    def zero_row(r, carry):
        for u in range(FULL16):
            out_v[r, pl.ds(u * 16, 16)] = zeros
        out_v[r, pl.ds(NUM_BINS - 16, 16)] = zeros
        return carry

    lax.fori_loop(0, CHUNK, zero_row, 0)


def _sc_scatter_body(y_hbm, out_hbm, y_v, out_v):
    wid = lax.axis_index("s") * NUM_CORES + lax.axis_index("c")
    row_base = wid * ROWS_PER_WORKER

    lane = lax.iota(jnp.int32, 16)
    zeros = jnp.zeros((16,), jnp.float32)

    _zero_fill(out_v)

    def chunk_body(ci, carry):
        yrow = (row_base + ci * CHUNK) // 128
        pltpu.sync_copy(y_hbm.at[pl.ds(yrow, 1)], y_v)

        touched = []
        for g in range(GROUPS):
            rid = g * 16 + lane
            y = y_v[0, pl.ds(g * 16, 16)]
            t = y + float(SUPPORT_RANGE)
            ti = jnp.clip(t.astype(jnp.int32), 0, NUM_BINS - 1)
            frac = t - ti.astype(jnp.float32)
            ihigh = jnp.minimum(ti + 1, NUM_BINS - 1)
            plsc.store_scatter(out_v, [rid, ti], 1.0 - frac)
            plsc.addupdate_scatter(out_v, [rid, ihigh], frac)
            touched.append((rid, ti, ihigh))

        pltpu.sync_copy(out_v, out_hbm.at[pl.ds(row_base + ci * CHUNK, CHUNK)])

        for rid, ti, ihigh in touched:
            plsc.store_scatter(out_v, [rid, ti], zeros)
            plsc.store_scatter(out_v, [rid, ihigh], zeros)
        return carry

    lax.fori_loop(0, N_CHUNKS, chunk_body, 0)


@jax.jit
def kernel(logits):
    y = pl.pallas_call(
        _row_scalar_block,
        grid=(N_ROWS // BLOCK_ROWS,),
        in_specs=[pl.BlockSpec((BLOCK_ROWS, NUM_BINS), lambda i: (i, 0))],
        out_specs=pl.BlockSpec((BLOCK_ROWS // 128, 128), lambda i: (i, 0)),
        out_shape=jax.ShapeDtypeStruct((N_ROWS // 128, 128), jnp.float32),
    )(logits)

    mesh = plsc.VectorSubcoreMesh(core_axis_name="c", subcore_axis_name="s")
    scatter = functools.partial(
        pl.kernel,
        mesh=mesh,
        out_type=jax.ShapeDtypeStruct((N_ROWS, NUM_BINS), jnp.float32),
        scratch_types=[
            pltpu.VMEM((1, CHUNK), jnp.float32),
            pltpu.VMEM((CHUNK, NUM_BINS), jnp.float32),
        ],
        compiler_params=pltpu.CompilerParams(needs_layout_passes=False),
    )(_sc_scatter_body)
    return scatter(y)
